# baseline (speedup 1.0000x reference)
import functools

import jax
import jax.numpy as jnp
from jax import lax
from jax.experimental import pallas as pl
from jax.experimental.pallas import tpu as pltpu
from jax.experimental.pallas import tpu_sc as plsc

B = 16384
D = 32
_NC = 2
_NS = 16
_NW = _NC * _NS
_BPW = B // _NW
_CHUNK = 128
_NCHUNK = _BPW // _CHUNK


def _gather_body(ids0, ids1, ids2, e0, e1, e2, o0, o1, o2,
                 idx0, idx1, idx2, r0, r1, r2, s0, s1, s2):
    w = lax.axis_index("s") * _NC + lax.axis_index("c")
    pltpu.sync_copy(ids0.at[w], idx0)
    pltpu.sync_copy(ids1.at[w], idx1)
    pltpu.sync_copy(ids2.at[w], idx2)
    waits = []
    for j in range(_NCHUNK):
        waits.append(pltpu.async_copy(e0.at[idx0.at[j]], r0.at[j], s0))
        waits.append(pltpu.async_copy(e1.at[idx1.at[j]], r1.at[j], s1))
        waits.append(pltpu.async_copy(e2.at[idx2.at[j]], r2.at[j], s2))
    for cp in waits:
        cp.wait()
    pltpu.sync_copy(r0, o0.at[w])
    pltpu.sync_copy(r1, o1.at[w])
    pltpu.sync_copy(r2, o2.at[w])


_row_t = jax.ShapeDtypeStruct((_NW, _NCHUNK, _CHUNK, D), jnp.float32)

_gather_call = functools.partial(
    pl.kernel,
    mesh=plsc.VectorSubcoreMesh(core_axis_name="c", subcore_axis_name="s"),
    compiler_params=pltpu.CompilerParams(use_tc_tiling_on_sc=False),
    out_type=(_row_t, _row_t, _row_t),
    scratch_types=[
        pltpu.VMEM((_NCHUNK, _CHUNK), jnp.int32),
        pltpu.VMEM((_NCHUNK, _CHUNK), jnp.int32),
        pltpu.VMEM((_NCHUNK, _CHUNK), jnp.int32),
        pltpu.VMEM((_NCHUNK, _CHUNK, D), jnp.float32),
        pltpu.VMEM((_NCHUNK, _CHUNK, D), jnp.float32),
        pltpu.VMEM((_NCHUNK, _CHUNK, D), jnp.float32),
        pltpu.SemaphoreType.DMA,
        pltpu.SemaphoreType.DMA,
        pltpu.SemaphoreType.DMA,
    ],
)(_gather_body)


def _proj_body(b0_ref, b1_ref, b2_ref, w10t_ref, w21t_ref, w12t_ref,
               bias_ref, o0_ref, o1_ref, o2_ref):
    b0 = b0_ref[...]
    b1 = b1_ref[...]
    b2 = b2_ref[...]
    o0_ref[...] = b0 + jnp.dot(b1, w10t_ref[...],
                               preferred_element_type=jnp.float32) + bias_ref[0, :]
    o1_ref[...] = b1 + jnp.dot(b2, w21t_ref[...],
                               preferred_element_type=jnp.float32) + bias_ref[1, :]
    o2_ref[...] = b2 + jnp.dot(b1, w12t_ref[...],
                               preferred_element_type=jnp.float32) + bias_ref[2, :]


_BLK = 2048
_out_t = jax.ShapeDtypeStruct((B, D), jnp.float32)

_proj_call = pl.pallas_call(
    _proj_body,
    grid=(B // _BLK,),
    in_specs=[
        pl.BlockSpec((_BLK, D), lambda i: (i, 0)),
        pl.BlockSpec((_BLK, D), lambda i: (i, 0)),
        pl.BlockSpec((_BLK, D), lambda i: (i, 0)),
        pl.BlockSpec((D, D), lambda i: (0, 0)),
        pl.BlockSpec((D, D), lambda i: (0, 0)),
        pl.BlockSpec((D, D), lambda i: (0, 0)),
        pl.BlockSpec((8, D), lambda i: (0, 0)),
    ],
    out_specs=[
        pl.BlockSpec((_BLK, D), lambda i: (i, 0)),
        pl.BlockSpec((_BLK, D), lambda i: (i, 0)),
        pl.BlockSpec((_BLK, D), lambda i: (i, 0)),
    ],
    out_shape=(_out_t, _out_t, _out_t),
)


def kernel(level_ids_0, level_ids_1, level_ids_2, emb0, emb1, emb2,
           W01, b01, W10, b10, W12, b12, W21, b21):
    del W01, b01
    ids0 = level_ids_0.astype(jnp.int32).reshape(_NW, _NCHUNK, _CHUNK)
    ids1 = level_ids_1.astype(jnp.int32).reshape(_NW, _NCHUNK, _CHUNK)
    ids2 = level_ids_2.astype(jnp.int32).reshape(_NW, _NCHUNK, _CHUNK)
    base0, base1, base2 = _gather_call(ids0, ids1, ids2, emb0, emb1, emb2)
    base0 = base0.reshape(B, D)
    base1 = base1.reshape(B, D)
    base2 = base2.reshape(B, D)
    bias = jnp.zeros((8, D), jnp.float32)
    bias = bias.at[0].set(b10).at[1].set(b21).at[2].set(b12)
    enh0, enh1, enh2 = _proj_call(base0, base1, base2, W10.T, W21.T, W12.T, bias)
    return (enh0, enh1, enh2)
